# baseline (speedup 1.0000x reference)
import jax
import jax.numpy as jnp
from jax import lax
from jax.experimental import pallas as pl
from jax.experimental.pallas import tpu as pltpu

N = 512
D = 128
E_DIM = 16
H = 64
BI = 32
NBLK = N // BI
LANES = BI * N
BIG = 1e30

_C00 = (((0,), (0,)), ((), ()))
_F32 = jnp.float32
_BF16 = jnp.bfloat16


def _mp_block(xt_ref, x_ref, et_ref, adj_ref, w1a_ref, w1b_ref, w1e_ref,
              selw_ref, selc_ref, b1_ref, w2_ref, b2_ref, u1x_ref,
              u1a_ref, c1_ref, u2_ref, c2_ref,
              out_ref, bmw_s, msumt_s, cntrow_s):
    i = pl.program_id(0)

    @pl.when(i == 0)
    def _init():
        bmt = lax.dot_general(w1b_ref[...], xt_ref[...], _C00,
                              preferred_element_type=_F32)
        bmw_s[...] = jnp.concatenate([bmt] * BI, axis=1)

    a_n = jnp.dot(x_ref[...], w1a_ref[...],
                  preferred_element_type=_F32) + b1_ref[...]
    aw = lax.dot_general(a_n.astype(_BF16), selw_ref[...], _C00,
                         preferred_element_type=_F32)

    m1 = (adj_ref[...] > 0).astype(_F32) - 1.0
    m1row = jnp.concatenate([m1[b:b + 1, :] for b in range(BI)], axis=1)
    moff = m1row * BIG

    ept = lax.dot_general(w1e_ref[...].astype(_BF16),
                          et_ref[...].astype(_BF16), _C00,
                          preferred_element_type=_F32)

    hm = jnp.maximum(ept + aw + bmw_s[...] + moff, 0.0)

    folds = [hm[:, b * N:b * N + 128] + hm[:, b * N + 128:b * N + 256]
             + hm[:, b * N + 256:b * N + 384] + hm[:, b * N + 384:(b + 1) * N]
             for b in range(BI)]
    hs = jnp.concatenate(folds, axis=1)
    msumt = jnp.dot(hs, selc_ref[...],
                    preferred_element_type=_F32)
    msumt_s[i] = msumt

    cnt_col = jnp.sum(m1, axis=1, keepdims=True) + float(N)
    cntrow_s[i] = lax.dot_general(
        cnt_col, jnp.eye(BI, dtype=_F32), _C00,
        preferred_element_type=_F32)

    @pl.when(i == NBLK - 1)
    def _final():
        cntrow = jnp.concatenate(
            [cntrow_s[k] for k in range(NBLK)], axis=1)
        msumt_all = jnp.concatenate(
            [msumt_s[k] for k in range(NBLK)], axis=1)
        degrow = jnp.maximum(cntrow, 1.0)
        aggt = (lax.dot_general(w2_ref[...], msumt_all, _C00,
                                preferred_element_type=_F32)
                + b2_ref[...] * cntrow) / degrow
        hidt = jnp.maximum(
            lax.dot_general(u1x_ref[...], xt_ref[...], _C00,
                            preferred_element_type=_F32)
            + lax.dot_general(u1a_ref[...], aggt, _C00,
                              preferred_element_type=_F32)
            + c1_ref[...], 0.0)
        out_ref[...] = (lax.dot_general(u2_ref[...], hidt, _C00,
                                        preferred_element_type=_F32)
                        + c2_ref[...])


def kernel(node_features, edge_features, adjacency, W1, b1, W2, b2, U1, c1,
           U2, c2):
    w1a = W1[:D]
    w1b = W1[D:2 * D]
    w1e = W1[2 * D:]
    selw = jnp.kron(jnp.eye(BI, dtype=_BF16), jnp.ones((1, N), _BF16))
    selc = jnp.kron(jnp.eye(BI, dtype=_F32), jnp.ones((128, 1), _F32))
    b1r = b1.reshape(1, H)
    b2c = b2.reshape(H, 1)
    c1c = c1.reshape(H, 1)
    c2c = c2.reshape(H, 1)

    et = edge_features.T
    xt = node_features.T

    full = lambda i: (0, 0)
    outt = pl.pallas_call(
        _mp_block,
        grid=(NBLK,),
        in_specs=[
            pl.BlockSpec((D, N), full),
            pl.BlockSpec((BI, D), lambda i: (i, 0)),
            pl.BlockSpec((E_DIM, LANES), lambda i: (0, i)),
            pl.BlockSpec((BI, N), lambda i: (i, 0)),
            pl.BlockSpec((D, H), full),
            pl.BlockSpec((D, H), full),
            pl.BlockSpec((E_DIM, H), full),
            pl.BlockSpec((BI, LANES), full),
            pl.BlockSpec((BI * 128, BI), full),
            pl.BlockSpec((1, H), full),
            pl.BlockSpec((H, H), full),
            pl.BlockSpec((H, 1), full),
            pl.BlockSpec((D, H), full),
            pl.BlockSpec((H, H), full),
            pl.BlockSpec((H, 1), full),
            pl.BlockSpec((H, H), full),
            pl.BlockSpec((H, 1), full),
        ],
        out_specs=pl.BlockSpec((H, N), full),
        out_shape=jax.ShapeDtypeStruct((H, N), _F32),
        scratch_shapes=[
            pltpu.VMEM((H, LANES), _F32),
            pltpu.VMEM((NBLK, H, BI), _F32),
            pltpu.VMEM((NBLK, 1, BI), _F32),
        ],
    )(xt, node_features, et, adjacency, w1a, w1b, w1e, selw, selc, b1r,
      W2, b2c, U1[:D], U1[D:], c1c, U2, c2c)
    return outt.T


# speedup vs baseline: 1.2761x; 1.2761x over previous
import jax
import jax.numpy as jnp
from jax import lax
from jax.experimental import pallas as pl
from jax.experimental.pallas import tpu as pltpu

N = 512
D = 128
E_DIM = 16
H = 64
BI = 32
NBLK = N // BI
LANES = BI * N
BIG = 1e30

_C00 = (((0,), (0,)), ((), ()))
_F32 = jnp.float32
_BF16 = jnp.bfloat16


def _mp_block(xt_ref, x_ref, et_ref, adj_ref, w1a_ref, w1b_ref, w1e_ref,
              selw_ref, selc_ref, b1_ref, w2_ref, b2_ref, u1x_ref,
              u1a_ref, c1_ref, u2_ref, c2_ref,
              out_ref, bmw_s, msumt_s, cntrow_s, comb_s):
    i = pl.program_id(0)

    @pl.when(i == 0)
    def _init():
        bmt = lax.dot_general(w1b_ref[...], xt_ref[...], _C00,
                              preferred_element_type=_F32)
        bmw_s[...] = jnp.concatenate([bmt] * BI, axis=1)
        comb_s[E_DIM:E_DIM + BI, :] = selw_ref[...]

    a_n = jnp.dot(x_ref[...], w1a_ref[...],
                  preferred_element_type=_F32) + b1_ref[...]

    m1 = (adj_ref[...] > 0).astype(_F32) - 1.0
    m1row = jnp.concatenate([m1[b:b + 1, :] for b in range(BI)], axis=1)

    comb_s[:E_DIM, :] = et_ref[...].astype(_BF16)
    comb_s[E_DIM + BI:, :] = m1row.astype(_BF16)
    lhs = jnp.concatenate(
        [w1e_ref[...].astype(_BF16), a_n.astype(_BF16),
         jnp.full((1, H), BIG, _BF16)], axis=0)
    pret = lax.dot_general(lhs, comb_s[...], _C00,
                           preferred_element_type=_F32)

    hm = jnp.maximum(pret + bmw_s[...], 0.0)

    folds = [hm[:, b * N:b * N + 128] + hm[:, b * N + 128:b * N + 256]
             + hm[:, b * N + 256:b * N + 384] + hm[:, b * N + 384:(b + 1) * N]
             for b in range(BI)]
    hs = jnp.concatenate(folds, axis=1)
    msumt = jnp.dot(hs, selc_ref[...],
                    preferred_element_type=_F32)
    msumt_s[i] = msumt

    cnt_col = jnp.sum(m1, axis=1, keepdims=True) + float(N)
    cntrow_s[i] = lax.dot_general(
        cnt_col, jnp.eye(BI, dtype=_F32), _C00,
        preferred_element_type=_F32)

    @pl.when(i == NBLK - 1)
    def _final():
        cntrow = jnp.concatenate(
            [cntrow_s[k] for k in range(NBLK)], axis=1)
        msumt_all = jnp.concatenate(
            [msumt_s[k] for k in range(NBLK)], axis=1)
        degrow = jnp.maximum(cntrow, 1.0)
        aggt = (lax.dot_general(w2_ref[...], msumt_all, _C00,
                                preferred_element_type=_F32)
                + b2_ref[...] * cntrow) / degrow
        hidt = jnp.maximum(
            lax.dot_general(u1x_ref[...], xt_ref[...], _C00,
                            preferred_element_type=_F32)
            + lax.dot_general(u1a_ref[...], aggt, _C00,
                              preferred_element_type=_F32)
            + c1_ref[...], 0.0)
        out_ref[...] = (lax.dot_general(u2_ref[...], hidt, _C00,
                                        preferred_element_type=_F32)
                        + c2_ref[...])


def kernel(node_features, edge_features, adjacency, W1, b1, W2, b2, U1, c1,
           U2, c2):
    w1a = W1[:D]
    w1b = W1[D:2 * D]
    w1e = W1[2 * D:]
    selw = jnp.kron(jnp.eye(BI, dtype=_BF16), jnp.ones((1, N), _BF16))
    selc = jnp.kron(jnp.eye(BI, dtype=_F32), jnp.ones((128, 1), _F32))
    b1r = b1.reshape(1, H)
    b2c = b2.reshape(H, 1)
    c1c = c1.reshape(H, 1)
    c2c = c2.reshape(H, 1)

    et = edge_features.T
    xt = node_features.T

    full = lambda i: (0, 0)
    outt = pl.pallas_call(
        _mp_block,
        grid=(NBLK,),
        in_specs=[
            pl.BlockSpec((D, N), full),
            pl.BlockSpec((BI, D), lambda i: (i, 0)),
            pl.BlockSpec((E_DIM, LANES), lambda i: (0, i)),
            pl.BlockSpec((BI, N), lambda i: (i, 0)),
            pl.BlockSpec((D, H), full),
            pl.BlockSpec((D, H), full),
            pl.BlockSpec((E_DIM, H), full),
            pl.BlockSpec((BI, LANES), full),
            pl.BlockSpec((BI * 128, BI), full),
            pl.BlockSpec((1, H), full),
            pl.BlockSpec((H, H), full),
            pl.BlockSpec((H, 1), full),
            pl.BlockSpec((D, H), full),
            pl.BlockSpec((H, H), full),
            pl.BlockSpec((H, 1), full),
            pl.BlockSpec((H, H), full),
            pl.BlockSpec((H, 1), full),
        ],
        out_specs=pl.BlockSpec((H, N), full),
        out_shape=jax.ShapeDtypeStruct((H, N), _F32),
        scratch_shapes=[
            pltpu.VMEM((H, LANES), _F32),
            pltpu.VMEM((NBLK, H, BI), _F32),
            pltpu.VMEM((NBLK, 1, BI), _F32),
            pltpu.VMEM((E_DIM + BI + 1, LANES), _BF16),
        ],
    )(xt, node_features, et, adjacency, w1a, w1b, w1e, selw, selc, b1r,
      W2, b2c, U1[:D], U1[D:], c1c, U2, c2c)
    return outt.T


# speedup vs baseline: 1.3780x; 1.0799x over previous
import jax
import jax.numpy as jnp
from jax import lax
from jax.experimental import pallas as pl
from jax.experimental.pallas import tpu as pltpu

N = 512
D = 128
E_DIM = 16
H = 64
BI = 64
NBLK = N // BI
LANES = BI * N
BIG = 1e30

_C00 = (((0,), (0,)), ((), ()))
_F32 = jnp.float32
_BF16 = jnp.bfloat16


def _mp_block(xt_ref, x_ref, et_ref, adj_ref, w1a_ref, w1b_ref, w1e_ref,
              selw_ref, selc_ref, b1_ref, w2_ref, b2_ref, u1x_ref,
              u1a_ref, c1_ref, u2_ref, c2_ref,
              out_ref, bmw_s, msumt_s, cntrow_s, comb_s):
    i = pl.program_id(0)

    @pl.when(i == 0)
    def _init():
        bmt = lax.dot_general(w1b_ref[...], xt_ref[...], _C00,
                              preferred_element_type=_F32).astype(_BF16)
        bmw_s[...] = jnp.concatenate([bmt] * BI, axis=1)
        comb_s[E_DIM:E_DIM + BI, :] = selw_ref[...]

    a_n = jnp.dot(x_ref[...], w1a_ref[...],
                  preferred_element_type=_F32) + b1_ref[...]

    m1 = (adj_ref[...] > 0).astype(_F32) - 1.0
    m1row = jnp.concatenate([m1[b:b + 1, :] for b in range(BI)], axis=1)

    comb_s[:E_DIM, :] = et_ref[...].astype(_BF16)
    comb_s[E_DIM + BI:, :] = m1row.astype(_BF16)
    lhs = jnp.concatenate(
        [w1e_ref[...].astype(_BF16), a_n.astype(_BF16),
         jnp.full((1, H), BIG, _BF16)], axis=0)
    pret = lax.dot_general(lhs, comb_s[...], _C00,
                           preferred_element_type=_F32)

    hm = jnp.maximum(pret + bmw_s[...], 0.0)

    folds = [hm[:, b * N:b * N + 128] + hm[:, b * N + 128:b * N + 256]
             + hm[:, b * N + 256:b * N + 384] + hm[:, b * N + 384:(b + 1) * N]
             for b in range(BI)]
    hs = jnp.concatenate(folds, axis=1).astype(_BF16)
    msumt = jnp.dot(hs, selc_ref[...],
                    preferred_element_type=_F32)
    msumt_s[i] = msumt

    cnt_col = jnp.sum(m1, axis=1, keepdims=True) + float(N)
    cntrow_s[i] = lax.dot_general(
        cnt_col, jnp.eye(BI, dtype=_F32), _C00,
        preferred_element_type=_F32)

    @pl.when(i == NBLK - 1)
    def _final():
        cntrow = jnp.concatenate(
            [cntrow_s[k] for k in range(NBLK)], axis=1)
        msumt_all = jnp.concatenate(
            [msumt_s[k] for k in range(NBLK)], axis=1)
        degrow = jnp.maximum(cntrow, 1.0)
        aggt = (lax.dot_general(w2_ref[...], msumt_all, _C00,
                                preferred_element_type=_F32)
                + b2_ref[...] * cntrow) / degrow
        hidt = jnp.maximum(
            lax.dot_general(u1x_ref[...], xt_ref[...], _C00,
                            preferred_element_type=_F32)
            + lax.dot_general(u1a_ref[...], aggt, _C00,
                              preferred_element_type=_F32)
            + c1_ref[...], 0.0)
        out_ref[...] = (lax.dot_general(u2_ref[...], hidt, _C00,
                                        preferred_element_type=_F32)
                        + c2_ref[...])


def kernel(node_features, edge_features, adjacency, W1, b1, W2, b2, U1, c1,
           U2, c2):
    w1a = W1[:D]
    w1b = W1[D:2 * D]
    w1e = W1[2 * D:]
    selw = jnp.kron(jnp.eye(BI, dtype=_BF16), jnp.ones((1, N), _BF16))
    selc = jnp.kron(jnp.eye(BI, dtype=_BF16), jnp.ones((128, 1), _BF16))
    b1r = b1.reshape(1, H)
    b2c = b2.reshape(H, 1)
    c1c = c1.reshape(H, 1)
    c2c = c2.reshape(H, 1)

    et = edge_features.T
    xt = node_features.T

    full = lambda i: (0, 0)
    outt = pl.pallas_call(
        _mp_block,
        grid=(NBLK,),
        in_specs=[
            pl.BlockSpec((D, N), full),
            pl.BlockSpec((BI, D), lambda i: (i, 0)),
            pl.BlockSpec((E_DIM, LANES), lambda i: (0, i)),
            pl.BlockSpec((BI, N), lambda i: (i, 0)),
            pl.BlockSpec((D, H), full),
            pl.BlockSpec((D, H), full),
            pl.BlockSpec((E_DIM, H), full),
            pl.BlockSpec((BI, LANES), full),
            pl.BlockSpec((BI * 128, BI), full),
            pl.BlockSpec((1, H), full),
            pl.BlockSpec((H, H), full),
            pl.BlockSpec((H, 1), full),
            pl.BlockSpec((D, H), full),
            pl.BlockSpec((H, H), full),
            pl.BlockSpec((H, 1), full),
            pl.BlockSpec((H, H), full),
            pl.BlockSpec((H, 1), full),
        ],
        out_specs=pl.BlockSpec((H, N), full),
        out_shape=jax.ShapeDtypeStruct((H, N), _F32),
        scratch_shapes=[
            pltpu.VMEM((H, LANES), _BF16),
            pltpu.VMEM((NBLK, H, BI), _F32),
            pltpu.VMEM((NBLK, 1, BI), _F32),
            pltpu.VMEM((E_DIM + BI + 1, LANES), _BF16),
        ],
    )(xt, node_features, et, adjacency, w1a, w1b, w1e, selw, selc, b1r,
      W2, b2c, U1[:D], U1[D:], c1c, U2, c2c)
    return outt.T
